# baseline (speedup 1.0000x reference)
import functools

import jax
import jax.numpy as jnp
from jax import lax
from jax.experimental import pallas as pl
from jax.experimental.pallas import tpu as pltpu
from jax.experimental.pallas import tpu_sc as plsc

NC, NS = 2, 16
NW = NC * NS
D = 1024
C = 32


def _pe_add(x2, ids3, weight, *, n_rows, steps):
    mesh = plsc.VectorSubcoreMesh(core_axis_name="c", subcore_axis_name="s")

    @functools.partial(
        pl.kernel,
        mesh=mesh,
        out_type=jax.ShapeDtypeStruct((n_rows, D), jnp.float32),
        scratch_types=[
            pltpu.VMEM((steps, C), jnp.int32),
            pltpu.VMEM((C, D), jnp.float32),
            pltpu.VMEM((C, D), jnp.float32),
            pltpu.VMEM((C, D), jnp.float32),
            pltpu.SemaphoreType.DMA,
            pltpu.SemaphoreType.DMA,
        ],
    )
    def k(x_hbm, ids_hbm, w_hbm, out_hbm, idx_v, bufx, bufw, bufo,
          semx, semw):
        wid = lax.axis_index("s") * NC + lax.axis_index("c")
        base = wid * (steps * C)

        pltpu.sync_copy(ids_hbm.at[wid], idx_v)

        def step(j, _):
            r0 = base + j * C
            cx = pltpu.async_copy(x_hbm.at[pl.ds(r0, C)], bufx, semx)
            cw = pltpu.async_copy(w_hbm.at[idx_v.at[j]], bufw, semw)
            cx.wait()
            cw.wait()

            @plsc.parallel_loop(0, C * (D // 16), unroll=4)
            def _add(i):
                r = i >> 6
                col = (i & (D // 16 - 1)) * 16
                bufo[r, pl.ds(col, 16)] = (
                    bufx[r, pl.ds(col, 16)] + bufw[r, pl.ds(col, 16)]
                )
            pltpu.sync_copy(bufo, out_hbm.at[pl.ds(r0, C)])
            return 0

        lax.fori_loop(0, steps, step, 0)

    return k(x2, ids3, weight)


def kernel(x, position_ids, weight):
    b, s, d = x.shape
    n_rows = b * s
    steps = n_rows // (NW * C)
    x2 = x.reshape(n_rows, d)
    ids3 = position_ids.reshape(NW, steps, C).astype(jnp.int32)
    out = _pe_add(x2, ids3, weight, n_rows=n_rows, steps=steps)
    return out.reshape(b, s, d)


# speedup vs baseline: 1.5303x; 1.5303x over previous
import jax
import jax.numpy as jnp
from jax import lax
from jax.experimental import pallas as pl
from jax.experimental.pallas import tpu as pltpu
from jax.experimental.pallas import tpu_sc as plsc

NC, NS = 2, 16
NW = NC * NS
D = 1024
C = 16


def _pe_add(x2, ids3, weight, *, n_rows, steps):
    mesh = plsc.VectorSubcoreMesh(core_axis_name="c", subcore_axis_name="s")

    @pl.kernel(
        mesh=mesh,
        out_type=jax.ShapeDtypeStruct((n_rows, D), jnp.float32),
        scratch_types=[
            pltpu.VMEM((steps, C), jnp.int32),
            [pltpu.VMEM((C, D), jnp.float32)] * 2,
            [pltpu.VMEM((C, D), jnp.float32)] * 2,
            [pltpu.VMEM((C, D), jnp.float32)] * 2,
            [pltpu.SemaphoreType.DMA] * 2,
            [pltpu.SemaphoreType.DMA] * 2,
            [pltpu.SemaphoreType.DMA] * 2,
        ],
    )
    def k(x_hbm, ids_hbm, w_hbm, out_hbm, idx_v, bufx, bufw, bufo,
          semx, semw, semo):
        wid = lax.axis_index("s") * NC + lax.axis_index("c")
        base = wid * (steps * C)

        pltpu.sync_copy(ids_hbm.at[wid], idx_v)

        def start_in(j, s):
            r0 = base + j * C
            pltpu.async_copy(x_hbm.at[pl.ds(r0, C)], bufx[s], semx[s])
            pltpu.async_copy(w_hbm.at[idx_v.at[j]], bufw[s], semw[s])

        def wait_out(s):
            pltpu.make_async_copy(bufo[s], out_hbm.at[pl.ds(0, C)],
                                  semo[s]).wait()

        start_in(0, 0)

        def outer(g, _):
            for p in (0, 1):
                j = g * 2 + p
                q = 1 - p
                pl.when(j + 1 < steps)(lambda: start_in(j + 1, q))
                pltpu.make_async_copy(x_hbm.at[pl.ds(0, C)], bufx[p],
                                      semx[p]).wait()
                pltpu.make_async_copy(w_hbm.at[pl.ds(0, C)], bufw[p],
                                      semw[p]).wait()
                pl.when(j >= 2)(lambda: wait_out(p))

                @plsc.parallel_loop(0, C * (D // 16), unroll=4)
                def _add(i):
                    r = i >> 6
                    col = (i & (D // 16 - 1)) * 16
                    bufo[p][r, pl.ds(col, 16)] = (
                        bufx[p][r, pl.ds(col, 16)] + bufw[p][r, pl.ds(col, 16)]
                    )

                pltpu.async_copy(bufo[p], out_hbm.at[pl.ds(base + j * C, C)],
                                 semo[p])
            return 0

        lax.fori_loop(0, steps // 2, outer, 0)
        wait_out(0)
        wait_out(1)

    return k(x2, ids3, weight)


def kernel(x, position_ids, weight):
    b, s, d = x.shape
    n_rows = b * s
    steps = n_rows // (NW * C)
    x2 = x.reshape(n_rows, d)
    ids3 = position_ids.reshape(NW, steps, C).astype(jnp.int32)
    out = _pe_add(x2, ids3, weight, n_rows=n_rows, steps=steps)
    return out.reshape(b, s, d)


# speedup vs baseline: 1.5344x; 1.0027x over previous
import jax
import jax.numpy as jnp
from jax import lax
from jax.experimental import pallas as pl
from jax.experimental.pallas import tpu as pltpu
from jax.experimental.pallas import tpu_sc as plsc

NC, NS = 2, 16
NW = NC * NS
D = 1024
C = 16


def _pe_add(x2, ids3, weight, *, n_rows, steps):
    mesh = plsc.VectorSubcoreMesh(core_axis_name="c", subcore_axis_name="s")

    @pl.kernel(
        mesh=mesh,
        out_type=jax.ShapeDtypeStruct((n_rows, D), jnp.float32),
        scratch_types=[
            pltpu.VMEM((steps, C), jnp.int32),
            [pltpu.VMEM((C, D), jnp.float32)] * 2,
            [pltpu.VMEM((C, D), jnp.float32)] * 2,
            [pltpu.VMEM((C, D), jnp.float32)] * 2,
            [pltpu.SemaphoreType.DMA] * 2,
            [pltpu.SemaphoreType.DMA] * 2,
            [pltpu.SemaphoreType.DMA] * 2,
        ],
    )
    def k(x_hbm, ids_hbm, w_hbm, out_hbm, idx_v, bufx, bufw, bufo,
          semx, semw, semo):
        wid = lax.axis_index("s") * NC + lax.axis_index("c")
        base = wid * (steps * C)

        pltpu.sync_copy(ids_hbm.at[wid], idx_v)

        def start_in(j, s):
            r0 = base + j * C
            pltpu.async_copy(x_hbm.at[pl.ds(r0, C)], bufx[s], semx[s])
            pltpu.async_copy(w_hbm.at[idx_v.at[j]], bufw[s], semw[s])

        def wait_out(s):
            pltpu.make_async_copy(bufo[s], out_hbm.at[pl.ds(0, C)],
                                  semo[s]).wait()

        start_in(0, 0)

        def outer(g, _):
            for p in (0, 1):
                j = g * 2 + p
                q = 1 - p
                pl.when(j + 1 < steps)(lambda: start_in(j + 1, q))
                pltpu.make_async_copy(x_hbm.at[pl.ds(0, C)], bufx[p],
                                      semx[p]).wait()
                pltpu.make_async_copy(w_hbm.at[pl.ds(0, C)], bufw[p],
                                      semw[p]).wait()
                pl.when(j >= 2)(lambda: wait_out(p))

                @plsc.parallel_loop(0, C * (D // 16), unroll=8)
                def _add(i):
                    r = i >> 6
                    col = (i & (D // 16 - 1)) * 16
                    bufo[p][r, pl.ds(col, 16)] = (
                        bufx[p][r, pl.ds(col, 16)] + bufw[p][r, pl.ds(col, 16)]
                    )

                pltpu.async_copy(bufo[p], out_hbm.at[pl.ds(base + j * C, C)],
                                 semo[p])
            return 0

        lax.fori_loop(0, steps // 2, outer, 0)
        wait_out(0)
        wait_out(1)

    return k(x2, ids3, weight)


def kernel(x, position_ids, weight):
    b, s, d = x.shape
    n_rows = b * s
    steps = n_rows // (NW * C)
    x2 = x.reshape(n_rows, d)
    ids3 = position_ids.reshape(NW, steps, C).astype(jnp.int32)
    out = _pe_add(x2, ids3, weight, n_rows=n_rows, steps=steps)
    return out.reshape(b, s, d)
